# baseline (speedup 1.0000x reference)
import functools

import jax
import jax.numpy as jnp
from jax import lax
from jax.experimental import pallas as pl
from jax.experimental.pallas import tpu as pltpu
from jax.experimental.pallas import tpu_sc as plsc

_N = 10000
_E = 320000
_D = 128
_K = 5

_NSUB = 16
_NCORE = 2
_CW = 128
_GW = 32
_NCHUNK = 8
_NBUF = 4
_LOOKAHEAD = 1
_NPAD = 10240
_NT = _NPAD // _NSUB
_DH = _D // _NCORE
_PAD_ROWS = 16


def _rsqrt_newton(d):
    bits = lax.bitcast_convert_type(d, jnp.int32)
    bits = jnp.int32(0x5F3759DF) - jnp.right_shift(bits, 1)
    y = lax.bitcast_convert_type(bits, jnp.float32)
    for _ in range(3):
        y = y * (1.5 - 0.5 * d * y * y)
    return y


def _sc_propagate(x_sp, row3, col3, *, npad, nt, cw, ch, dh, kk, interpret=False):
    mesh = plsc.VectorSubcoreMesh(
        core_axis_name="core", subcore_axis_name="subcore",
        num_cores=_NCORE, num_subcores=_NSUB)

    gw = _GW
    chg = ch // gw
    nc = nt // _NCHUNK

    @functools.partial(
        pl.kernel,
        out_type=(
            jax.ShapeDtypeStruct((kk, 2, npad, dh), jnp.float32),
            jax.ShapeDtypeStruct((npad,), jnp.float32),
        ),
        mesh=mesh,
        scratch_types=[
            pltpu.VMEM((gw, cw), jnp.int32),
            pltpu.VMEM((gw, cw), jnp.int32),
            *[pltpu.VMEM((cw, dh), jnp.float32) for _ in range(_NBUF)],
            pltpu.VMEM((nt // _NCHUNK, dh), jnp.float32),
            pltpu.VMEM((nt,), jnp.float32),
            pltpu.VMEM((nt,), jnp.float32),
            pltpu.VMEM_SHARED((npad, dh), jnp.float32),
            pltpu.VMEM_SHARED((npad, dh), jnp.float32),
            pltpu.VMEM_SHARED((npad,), jnp.float32),
            *[pltpu.SemaphoreType.DMA for _ in range(2 * _NBUF)],
            pltpu.SemaphoreType.DMA,
        ],
        compiler_params=pltpu.CompilerParams(use_tc_tiling_on_sc=False),
        interpret=interpret,
    )
    def body(x_hbm, row_hbm, col_hbm, hs_hbm, dis_hbm,
             rowg, colg, *rest):
        bufs = rest[:_NBUF]
        (nodebuf, dis_v, dtmp, g_sh, s_sh, deg_sh) = rest[_NBUF:_NBUF + 6]
        gsems = rest[_NBUF + 6:2 * _NBUF + 6]
        ssems = rest[2 * _NBUF + 6:3 * _NBUF + 6]
        semd = rest[3 * _NBUF + 6]
        c = lax.axis_index("core")
        s = lax.axis_index("subcore")
        nbase = s * nt
        nsl = pl.ds(nbase, nt)

        @pl.loop(0, nt // 16)
        def _(i):
            dtmp[pl.ds(i * 16, 16)] = jnp.zeros((16,), jnp.float32)

        pltpu.sync_copy(dtmp, deg_sh.at[nsl])

        @pl.loop(0, cw // 16)
        def _(i):
            dtmp[pl.ds(i * 16, 16)] = jnp.ones((16,), jnp.float32)

        plsc.subcore_barrier()

        vone = dtmp.at[pl.ds(0, cw)]

        @pl.loop(0, chg)
        def _(gi):
            pltpu.sync_copy(col_hbm.at[s, pl.ds(gi * gw, gw)], colg)
            descs = [
                pltpu.async_copy(vone, deg_sh.at[colg.at[jj]], semd, add=True)
                for jj in range(gw)
            ]
            for dsc in descs:
                dsc.wait()

        plsc.subcore_barrier()

        pltpu.sync_copy(deg_sh.at[nsl], dtmp)

        @pl.loop(0, nt // 16)
        def _(i):
            sl = pl.ds(i * 16, 16)
            dis_v[sl] = _rsqrt_newton(dtmp[sl] + 1.0)

        def scale_chunk(ci):
            @pl.loop(0, nc // 16)
            def _(i16):
                dvec = dis_v[pl.ds(ci * nc + i16 * 16, 16)]
                for l in range(16):
                    dv = dvec[l]
                    for r in range(dh // 16):
                        sl = pl.ds(r * 16, 16)
                        nodebuf[i16 * 16 + l, sl] = nodebuf[i16 * 16 + l, sl] * dv

        for ci in range(_NCHUNK):
            csl = pl.ds(nbase + ci * nc, nc)
            pltpu.sync_copy(x_hbm.at[c, csl], nodebuf)
            scale_chunk(ci)
            pltpu.sync_copy(nodebuf, g_sh.at[csl])
            pltpu.sync_copy(nodebuf, s_sh.at[csl])

        pltpu.sync_copy(dis_v, dis_hbm.at[nsl])

        @pl.loop(0, nt // 16)
        def _(i):
            sl = pl.ds(i * 16, 16)
            v = dis_v[sl]
            dis_v[sl] = v * v

        plsc.subcore_barrier()

        @pl.loop(0, kk)
        def _(k):
            @pl.loop(0, chg)
            def _(gi):
                gsl = pl.ds(gi * gw, gw)
                pltpu.sync_copy(row_hbm.at[s, gsl], rowg)
                pltpu.sync_copy(col_hbm.at[s, gsl], colg)
                gd = [None] * _NBUF
                pend = [None] * _NBUF
                for q in range(min(_LOOKAHEAD, gw)):
                    gd[q % _NBUF] = pltpu.async_copy(
                        g_sh.at[rowg.at[q]], bufs[q % _NBUF], gsems[q % _NBUF])
                for m in range(gw):
                    q = m + _LOOKAHEAD
                    if q < gw:
                        b2 = q % _NBUF
                        if pend[b2] is not None:
                            pend[b2].wait()
                            pend[b2] = None
                        gd[b2] = pltpu.async_copy(
                            g_sh.at[rowg.at[q]], bufs[b2], gsems[b2])
                    b = m % _NBUF
                    gd[b].wait()
                    pend[b] = pltpu.async_copy(
                        bufs[b], s_sh.at[colg.at[m]], ssems[b], add=True)
                for b in range(_NBUF):
                    if pend[b] is not None:
                        pend[b].wait()

            plsc.subcore_barrier()

            pend_gs = []
            for ci in range(_NCHUNK):
                csl = pl.ds(nbase + ci * nc, nc)
                d_hs = pltpu.async_copy(
                    s_sh.at[csl], hs_hbm.at[k, c, csl], gsems[0])
                for dsc in pend_gs:
                    dsc.wait()
                pltpu.sync_copy(s_sh.at[csl], nodebuf)
                scale_chunk(ci)
                d_hs.wait()
                pend_gs = [
                    pltpu.async_copy(nodebuf, g_sh.at[csl], ssems[0]),
                    pltpu.async_copy(nodebuf, s_sh.at[csl], ssems[1]),
                ]
            for dsc in pend_gs:
                dsc.wait()

            plsc.subcore_barrier()

    return body(x_sp, row3, col3)


def _tc_combine(x, hs, dis, proj_w, proj_b, *, n, d, kk, bn, interpret=False):
    def body(x_ref, hs_ref, dis_ref, w_ref, b_ref, o_ref):
        x = x_ref[...]
        dv = dis_ref[...]
        w8 = jnp.broadcast_to(w_ref[...], (8, d))
        b = b_ref[0, 0]

        def retain(m):
            z = jax.lax.dot_general(
                m, w8, (((1,), (1,)), ((), ())),
                preferred_element_type=jnp.float32)[:, 0:1] + b
            return jax.nn.sigmoid(z)

        acc = retain(x) * x
        for k in range(kk):
            hk = dv * jnp.concatenate([hs_ref[k, 0], hs_ref[k, 1]], axis=-1)
            acc = acc + retain(hk) * hk
        o_ref[...] = acc

    return pl.pallas_call(
        body,
        out_shape=jax.ShapeDtypeStruct((n, d), jnp.float32),
        grid=(n // bn,),
        in_specs=[
            pl.BlockSpec((bn, d), lambda i: (i, 0)),
            pl.BlockSpec((kk, 2, bn, d // 2), lambda i: (0, 0, i, 0)),
            pl.BlockSpec((bn, 1), lambda i: (i, 0)),
            pl.BlockSpec((1, d), lambda i: (0, 0)),
            pl.BlockSpec((1, 1), lambda i: (0, 0)),
        ],
        out_specs=pl.BlockSpec((bn, d), lambda i: (i, 0)),
        interpret=interpret,
    )(x, hs, dis, proj_w, proj_b.reshape(1, 1))


def _run(x, edge_index, proj_w, proj_b, *, npad, cw, interpret=False):
    n, d = x.shape
    e = edge_index.shape[1]
    nsub = _NSUB
    etile = -(-e // nsub)
    ch = -(-etile // cw)
    ch = -(-ch // _GW) * _GW
    epad = nsub * ch * cw - e

    dh = d // _NCORE
    x_sp = (jnp.zeros((_NCORE, npad, dh), jnp.float32)
            .at[0, :n].set(x[:, :dh]).at[1, :n].set(x[:, dh:]))
    row = edge_index[0]
    col = edge_index[1]
    padrow = jnp.zeros((epad,), jnp.int32)
    padcol = n + (jnp.arange(epad, dtype=jnp.int32) % _PAD_ROWS)
    row3 = jnp.concatenate([row, padrow]).reshape(nsub, ch, cw)
    col3 = jnp.concatenate([col, padcol]).reshape(nsub, ch, cw)

    hs, dis = _sc_propagate(
        x_sp, row3, col3,
        npad=npad, nt=npad // nsub, cw=cw, ch=ch, dh=dh, kk=_K,
        interpret=interpret)
    return _tc_combine(
        x, hs, dis.reshape(npad, 1), proj_w, proj_b,
        n=n, d=d, kk=_K, bn=2000, interpret=interpret)


def kernel(x, edge_index, proj_w, proj_b):
    return _run(x, edge_index, proj_w, proj_b, npad=_NPAD, cw=_CW)


# speedup vs baseline: 1.0053x; 1.0053x over previous
import functools

import jax
import jax.numpy as jnp
from jax import lax
from jax.experimental import pallas as pl
from jax.experimental.pallas import tpu as pltpu
from jax.experimental.pallas import tpu_sc as plsc

_N = 10000
_E = 320000
_D = 128
_K = 5

_NSUB = 16
_NCORE = 2
_CW = 128
_GW = 32
_NCHUNK = 4
_NBUF = 3
_LOOKAHEAD = 1
_NPAD = 10240
_NT = _NPAD // _NSUB
_DH = _D // _NCORE
_PAD_ROWS = 16


def _rsqrt_newton(d):
    bits = lax.bitcast_convert_type(d, jnp.int32)
    bits = jnp.int32(0x5F3759DF) - jnp.right_shift(bits, 1)
    y = lax.bitcast_convert_type(bits, jnp.float32)
    for _ in range(3):
        y = y * (1.5 - 0.5 * d * y * y)
    return y


def _sc_propagate(x_sp, row3, col3, *, npad, nt, cw, ch, dh, kk, interpret=False):
    mesh = plsc.VectorSubcoreMesh(
        core_axis_name="core", subcore_axis_name="subcore",
        num_cores=_NCORE, num_subcores=_NSUB)

    gw = _GW
    chg = ch // gw
    nc = nt // _NCHUNK

    @functools.partial(
        pl.kernel,
        out_type=(
            jax.ShapeDtypeStruct((kk, 2, npad, dh), jnp.float32),
            jax.ShapeDtypeStruct((npad,), jnp.float32),
        ),
        mesh=mesh,
        scratch_types=[
            pltpu.VMEM((gw, cw), jnp.int32),
            pltpu.VMEM((gw, cw), jnp.int32),
            *[pltpu.VMEM((cw, dh), jnp.float32) for _ in range(_NBUF)],
            pltpu.VMEM((nt // _NCHUNK, dh), jnp.float32),
            pltpu.VMEM((nt,), jnp.float32),
            pltpu.VMEM((nt,), jnp.float32),
            pltpu.VMEM_SHARED((npad, dh), jnp.float32),
            pltpu.VMEM_SHARED((npad, dh), jnp.float32),
            pltpu.VMEM_SHARED((npad,), jnp.float32),
            *[pltpu.SemaphoreType.DMA for _ in range(2 * _NBUF)],
            pltpu.SemaphoreType.DMA,
        ],
        compiler_params=pltpu.CompilerParams(use_tc_tiling_on_sc=False),
        interpret=interpret,
    )
    def body(x_hbm, row_hbm, col_hbm, hs_hbm, dis_hbm,
             rowg, colg, *rest):
        bufs = rest[:_NBUF]
        (nodebuf, dis_v, dtmp, g_sh, s_sh, deg_sh) = rest[_NBUF:_NBUF + 6]
        gsems = rest[_NBUF + 6:2 * _NBUF + 6]
        ssems = rest[2 * _NBUF + 6:3 * _NBUF + 6]
        semd = rest[3 * _NBUF + 6]
        c = lax.axis_index("core")
        s = lax.axis_index("subcore")
        nbase = s * nt
        nsl = pl.ds(nbase, nt)

        @pl.loop(0, nt // 16)
        def _(i):
            dtmp[pl.ds(i * 16, 16)] = jnp.zeros((16,), jnp.float32)

        pltpu.sync_copy(dtmp, deg_sh.at[nsl])

        @pl.loop(0, cw // 16)
        def _(i):
            dtmp[pl.ds(i * 16, 16)] = jnp.ones((16,), jnp.float32)

        plsc.subcore_barrier()

        vone = dtmp.at[pl.ds(0, cw)]

        @pl.loop(0, chg)
        def _(gi):
            pltpu.sync_copy(col_hbm.at[s, pl.ds(gi * gw, gw)], colg)
            descs = [
                pltpu.async_copy(vone, deg_sh.at[colg.at[jj]], semd, add=True)
                for jj in range(gw)
            ]
            for dsc in descs:
                dsc.wait()

        plsc.subcore_barrier()

        pltpu.sync_copy(deg_sh.at[nsl], dtmp)

        @pl.loop(0, nt // 16)
        def _(i):
            sl = pl.ds(i * 16, 16)
            dis_v[sl] = _rsqrt_newton(dtmp[sl] + 1.0)

        def scale_chunk(ci):
            @pl.loop(0, nc // 16)
            def _(i16):
                dvec = dis_v[pl.ds(ci * nc + i16 * 16, 16)]
                for l in range(16):
                    dv = dvec[l]
                    for r in range(dh // 16):
                        sl = pl.ds(r * 16, 16)
                        nodebuf[i16 * 16 + l, sl] = nodebuf[i16 * 16 + l, sl] * dv

        for ci in range(_NCHUNK):
            csl = pl.ds(nbase + ci * nc, nc)
            pltpu.sync_copy(x_hbm.at[c, csl], nodebuf)
            scale_chunk(ci)
            pltpu.sync_copy(nodebuf, g_sh.at[csl])
            pltpu.sync_copy(nodebuf, s_sh.at[csl])

        pltpu.sync_copy(dis_v, dis_hbm.at[nsl])

        @pl.loop(0, nt // 16)
        def _(i):
            sl = pl.ds(i * 16, 16)
            v = dis_v[sl]
            dis_v[sl] = v * v

        plsc.subcore_barrier()

        @pl.loop(0, kk)
        def _(k):
            @pl.loop(0, chg)
            def _(gi):
                gsl = pl.ds(gi * gw, gw)
                pltpu.sync_copy(row_hbm.at[s, gsl], rowg)
                pltpu.sync_copy(col_hbm.at[s, gsl], colg)
                gd = [None] * _NBUF
                pend = [None] * _NBUF
                for q in range(min(_LOOKAHEAD, gw)):
                    gd[q % _NBUF] = pltpu.async_copy(
                        g_sh.at[rowg.at[q]], bufs[q % _NBUF], gsems[q % _NBUF])
                for m in range(gw):
                    q = m + _LOOKAHEAD
                    if q < gw:
                        b2 = q % _NBUF
                        if pend[b2] is not None:
                            pend[b2].wait()
                            pend[b2] = None
                        gd[b2] = pltpu.async_copy(
                            g_sh.at[rowg.at[q]], bufs[b2], gsems[b2])
                    b = m % _NBUF
                    gd[b].wait()
                    pend[b] = pltpu.async_copy(
                        bufs[b], s_sh.at[colg.at[m]], ssems[b], add=True)
                for b in range(_NBUF):
                    if pend[b] is not None:
                        pend[b].wait()

            plsc.subcore_barrier()

            pend_gs = []
            for ci in range(_NCHUNK):
                csl = pl.ds(nbase + ci * nc, nc)
                d_hs = pltpu.async_copy(
                    s_sh.at[csl], hs_hbm.at[k, c, csl], gsems[0])
                for dsc in pend_gs:
                    dsc.wait()
                pltpu.sync_copy(s_sh.at[csl], nodebuf)
                scale_chunk(ci)
                d_hs.wait()
                pend_gs = [
                    pltpu.async_copy(nodebuf, g_sh.at[csl], ssems[0]),
                    pltpu.async_copy(nodebuf, s_sh.at[csl], ssems[1]),
                ]
            for dsc in pend_gs:
                dsc.wait()

            plsc.subcore_barrier()

    return body(x_sp, row3, col3)


def _tc_combine(x, hs, dis, proj_w, proj_b, *, n, d, kk, bn, interpret=False):
    def body(x_ref, hs_ref, dis_ref, w_ref, b_ref, o_ref):
        x = x_ref[...]
        dv = dis_ref[...]
        w8 = jnp.broadcast_to(w_ref[...], (8, d))
        b = b_ref[0, 0]

        def retain(m):
            z = jax.lax.dot_general(
                m, w8, (((1,), (1,)), ((), ())),
                preferred_element_type=jnp.float32)[:, 0:1] + b
            return jax.nn.sigmoid(z)

        acc = retain(x) * x
        for k in range(kk):
            hk = dv * jnp.concatenate([hs_ref[k, 0], hs_ref[k, 1]], axis=-1)
            acc = acc + retain(hk) * hk
        o_ref[...] = acc

    return pl.pallas_call(
        body,
        out_shape=jax.ShapeDtypeStruct((n, d), jnp.float32),
        grid=(n // bn,),
        in_specs=[
            pl.BlockSpec((bn, d), lambda i: (i, 0)),
            pl.BlockSpec((kk, 2, bn, d // 2), lambda i: (0, 0, i, 0)),
            pl.BlockSpec((bn, 1), lambda i: (i, 0)),
            pl.BlockSpec((1, d), lambda i: (0, 0)),
            pl.BlockSpec((1, 1), lambda i: (0, 0)),
        ],
        out_specs=pl.BlockSpec((bn, d), lambda i: (i, 0)),
        interpret=interpret,
    )(x, hs, dis, proj_w, proj_b.reshape(1, 1))


def _run(x, edge_index, proj_w, proj_b, *, npad, cw, interpret=False):
    n, d = x.shape
    e = edge_index.shape[1]
    nsub = _NSUB
    etile = -(-e // nsub)
    ch = -(-etile // cw)
    ch = -(-ch // _GW) * _GW
    epad = nsub * ch * cw - e

    dh = d // _NCORE
    x_sp = (jnp.zeros((_NCORE, npad, dh), jnp.float32)
            .at[0, :n].set(x[:, :dh]).at[1, :n].set(x[:, dh:]))
    row = edge_index[0]
    col = edge_index[1]
    padrow = jnp.zeros((epad,), jnp.int32)
    padcol = n + (jnp.arange(epad, dtype=jnp.int32) % _PAD_ROWS)
    row3 = jnp.concatenate([row, padrow]).reshape(nsub, ch, cw)
    col3 = jnp.concatenate([col, padcol]).reshape(nsub, ch, cw)

    hs, dis = _sc_propagate(
        x_sp, row3, col3,
        npad=npad, nt=npad // nsub, cw=cw, ch=ch, dh=dh, kk=_K,
        interpret=interpret)
    return _tc_combine(
        x, hs, dis.reshape(npad, 1), proj_w, proj_b,
        n=n, d=d, kk=_K, bn=2000, interpret=interpret)


def kernel(x, edge_index, proj_w, proj_b):
    return _run(x, edge_index, proj_w, proj_b, npad=_NPAD, cw=_CW)
